# Initial kernel scaffold; baseline (speedup 1.0000x reference)
import jax
import jax.numpy as jnp
from jax.experimental import pallas as pl


def kernel(x, edge_index, batch, gc_content, seq_length, Wq0, bq0, Wk0, bk0, Wv0, bv0, Ws0, bs0, g0, b0, Wq1, bq1, Wk1, bk1, Wv1, bv1, Ws1, bs1, g1, b1, Wq2, bq2, Wk2, bk2, Wv2, bv2, Ws2, bs2, g2, b2, fcW, fcb, emb, Wih, Whh, bih, bhh, outW, outb):
    raise NotImplementedError("write your pallas kernel here")


# speedup vs baseline: 1.0003x; 1.0003x over previous
import jax
import jax.numpy as jnp
from jax.experimental import pallas as pl
from jax.experimental.pallas import tpu as pltpu

N = 10000
E = 160000
B = 512
H = 2
OC = 128
D = H * OC
ED = 64
V = 5
T = 21
EPS = 1e-5


def _decoder_body(gf_ref, fcW_ref, fcb_ref, emb_ref, Wih_ref, Whh_ref,
                  bihh_ref, outW_ref, outb_ref, out_ref):
    gf = gf_ref[...]
    enc = jax.nn.relu(
        jnp.dot(gf, fcW_ref[...], preferred_element_type=jnp.float32)
        + fcb_ref[...])
    emb = emb_ref[...]
    Wih = Wih_ref[...]
    Whh = Whh_ref[...]
    bihh = bihh_ref[...]
    outW = outW_ref[...]
    outb = outb_ref[...]
    hs = enc
    cs = jnp.zeros_like(enc)
    inp = jnp.broadcast_to(emb[1], (B, ED))
    for t in range(T):
        gates = (jnp.dot(inp, Wih, preferred_element_type=jnp.float32)
                 + jnp.dot(hs, Whh, preferred_element_type=jnp.float32)
                 + bihh)
        i_ = gates[:, 0 * ED:1 * ED]
        f_ = gates[:, 1 * ED:2 * ED]
        g_ = gates[:, 2 * ED:3 * ED]
        o_ = gates[:, 3 * ED:4 * ED]
        cs = jax.nn.sigmoid(f_) * cs + jax.nn.sigmoid(i_) * jnp.tanh(g_)
        hs = jax.nn.sigmoid(o_) * jnp.tanh(cs)
        logits = (jnp.dot(hs, outW, preferred_element_type=jnp.float32)
                  + outb)
        out_ref[:, t, :] = logits
        m = jnp.max(logits, axis=-1, keepdims=True)
        iota = jax.lax.broadcasted_iota(jnp.int32, (B, V), 1)
        tok = jnp.min(jnp.where(logits == m, iota, V), axis=-1,
                      keepdims=True)
        onehot = (jax.lax.broadcasted_iota(jnp.int32, (B, V), 1)
                  == tok).astype(jnp.float32)
        inp = jnp.dot(onehot, emb, preferred_element_type=jnp.float32)


def _decode(gf, fcW, fcb, emb, Wih, Whh, bihh, outW, outb):
    return pl.pallas_call(
        _decoder_body,
        out_shape=jax.ShapeDtypeStruct((B, T, V), jnp.float32),
    )(gf, fcW, fcb, emb, Wih, Whh, bihh, outW, outb)


def _tconv(x, src, dst, Wq, bq, Wk, bk, Wv, bv, Ws, bs):
    n = x.shape[0]
    q = (x @ Wq + bq)[dst].reshape(-1, H, OC)
    k = (x @ Wk + bk)[src].reshape(-1, H, OC)
    v = (x @ Wv + bv)[src].reshape(-1, H, OC)
    alpha = (q * k).sum(-1) / (OC ** 0.5)
    amax = jax.ops.segment_max(alpha, dst, num_segments=n)
    amax = jnp.where(jnp.isfinite(amax), amax, 0.0)
    ex = jnp.exp(alpha - amax[dst])
    den = jax.ops.segment_sum(ex, dst, num_segments=n)
    w = ex / (den[dst] + 1e-16)
    agg = jax.ops.segment_sum(v * w[..., None], dst, num_segments=n)
    return agg.reshape(n, D) + x @ Ws + bs


def _ln(x, g, b):
    mu = jnp.mean(x, axis=-1, keepdims=True)
    var = jnp.var(x, axis=-1, keepdims=True)
    return (x - mu) / jnp.sqrt(var + EPS) * g + b


def kernel(x, edge_index, batch, gc_content, seq_length,
           Wq0, bq0, Wk0, bk0, Wv0, bv0, Ws0, bs0, g0, b0,
           Wq1, bq1, Wk1, bk1, Wv1, bv1, Ws1, bs1, g1, b1,
           Wq2, bq2, Wk2, bk2, Wv2, bv2, Ws2, bs2, g2, b2,
           fcW, fcb, emb, Wih, Whh, bih, bhh, outW, outb):
    src, dst = edge_index[0], edge_index[1]
    layers = [
        (Wq0, bq0, Wk0, bk0, Wv0, bv0, Ws0, bs0, g0, b0),
        (Wq1, bq1, Wk1, bk1, Wv1, bv1, Ws1, bs1, g1, b1),
        (Wq2, bq2, Wk2, bk2, Wv2, bv2, Ws2, bs2, g2, b2),
    ]
    h = x
    for (Wq, bq, Wk, bk, Wv, bv, Ws, bs, g, b) in layers:
        h = jax.nn.relu(_tconv(h, src, dst, Wq, bq, Wk, bk, Wv, bv, Ws, bs))
        h = _ln(h, g, b)
    cnt = jax.ops.segment_sum(jnp.ones((N,), jnp.float32), batch,
                              num_segments=B)
    pooled = (jax.ops.segment_sum(h, batch, num_segments=B)
              / jnp.maximum(cnt, 1.0)[:, None])
    gf = jnp.concatenate(
        [pooled, gc_content[:, None], seq_length[:, None]], axis=1)
    return _decode(gf, fcW, fcb, emb, Wih, Whh, bih + bhh, outW, outb)


# speedup vs baseline: 23.1134x; 23.1067x over previous
import functools

import jax
import jax.numpy as jnp
from jax import lax
from jax.experimental import pallas as pl
from jax.experimental.pallas import tpu as pltpu
from jax.experimental.pallas import tpu_sc as plsc

N = 10000
NP = 10240
E = 160000
B = 512
H = 2
OC = 128
D = H * OC
ED = 64
V = 5
T = 21
EPS = 1e-5

EP = E // 16
CH = 80
NCH = EP // CH
NPT = NP // 16
SCALE = 1.0 / (OC ** 0.5)


def _edge_body(qh, kh, vh, srch, dsth, out, alout,
               qidx, kidx, dstb, tmp80, arow, qbuf, kbuf, exrow,
               den, dacc, dtmp, vmax16, mbuf, aggbuf,
               agg_s, dens_s, maxes_s, sem0, sem1):
    c = lax.axis_index("c")
    w = lax.axis_index("s")
    ebase = w * EP
    nbase = w * NPT
    zero16 = jnp.zeros((16,), jnp.float32)
    izero = jnp.zeros((16,), jnp.int32)
    lane = lax.iota(jnp.int32, 16)

    def bcast_lane(vec, r):
        dnums = lax.GatherDimensionNumbers(
            offset_dims=(), collapsed_slice_dims=(0,), start_index_map=(0,))
        return lax.gather(vec, (izero + r)[:, None], dnums, (1,),
                          mode=lax.GatherScatterMode.PROMISE_IN_BOUNDS)

    def z_qbuf(r, _):
        for jj in range(8):
            qbuf[r, pl.ds(jj * 16, 16)] = zero16
        return 0
    lax.fori_loop(0, CH, z_qbuf, 0)
    for t in range(NPT // CH):
        pltpu.sync_copy(qbuf, agg_s.at[pl.ds(pl.multiple_of(nbase + t * CH, 16), CH)])

    def z_den(i, _):
        den[pl.ds(i * 16, 16)] = zero16
        return 0
    lax.fori_loop(0, NP // 16, z_den, 0)

    def p1_chunk(j, runmax):
        eb = ebase + j * CH
        pltpu.sync_copy(dsth.at[pl.ds(eb, CH)], tmp80)

        def idx_d(i, _):
            d16 = tmp80[pl.ds(i * 16, 16)]
            qidx[pl.ds(i * 16, 16)] = d16 * 2 + c
            return 0
        lax.fori_loop(0, CH // 16, idx_d, 0)
        pltpu.sync_copy(srch.at[pl.ds(eb, CH)], tmp80)

        def idx_s(i, _):
            s16 = tmp80[pl.ds(i * 16, 16)]
            kidx[pl.ds(i * 16, 16)] = s16 * 2 + c
            return 0
        lax.fori_loop(0, CH // 16, idx_s, 0)

        cq = pltpu.async_copy(qh.at[qidx], qbuf, sem0)
        ck = pltpu.async_copy(kh.at[kidx], kbuf, sem1)
        cq.wait()
        ck.wait()

        def grp(g, rm):
            def edg(e16, carry):
                a16, rmi = carry
                e = g * 16 + e16
                acc = qbuf[e, pl.ds(0, 16)] * kbuf[e, pl.ds(0, 16)]
                for jj in range(1, 8):
                    acc = acc + (qbuf[e, pl.ds(jj * 16, 16)]
                                 * kbuf[e, pl.ds(jj * 16, 16)])
                aval = jnp.sum(acc) * SCALE
                a16 = jnp.where(lane == e16, aval, a16)
                return (a16, rmi)
            a16, rm = lax.fori_loop(0, 16, edg, (zero16, rm))
            arow[pl.ds(g * 16, 16)] = a16
            return jnp.maximum(rm, a16)
        runmax = lax.fori_loop(0, CH // 16, grp, runmax)
        pltpu.sync_copy(arow, alout.at[pl.ds(pl.multiple_of(c * E + eb, 16), CH)])
        return runmax

    runmax = lax.fori_loop(0, NCH, p1_chunk,
                           jnp.full((16,), -1e30, jnp.float32))
    vmax16[pl.ds(0, 16)] = runmax
    pltpu.sync_copy(vmax16, maxes_s.at[pl.ds(pl.multiple_of(w * 16, 16), 16)])
    plsc.subcore_barrier()

    pltpu.sync_copy(maxes_s, mbuf)
    m = mbuf[pl.ds(0, 16)]
    for t in range(1, 16):
        m = jnp.maximum(m, mbuf[pl.ds(t * 16, 16)])
    gmax = jnp.max(m)

    def p2_chunk(j, _):
        eb = ebase + j * CH
        pltpu.sync_copy(srch.at[pl.ds(eb, CH)], tmp80)

        def idx_s(i, _):
            s16 = tmp80[pl.ds(i * 16, 16)]
            kidx[pl.ds(i * 16, 16)] = s16 * 2 + c
            return 0
        lax.fori_loop(0, CH // 16, idx_s, 0)
        cv = pltpu.async_copy(vh.at[kidx], qbuf, sem0)
        pltpu.sync_copy(dsth.at[pl.ds(eb, CH)], tmp80)

        def idx_d(i, _):
            dstb[0, pl.ds(i * 16, 16)] = tmp80[pl.ds(i * 16, 16)]
            return 0
        lax.fori_loop(0, CH // 16, idx_d, 0)
        pltpu.sync_copy(alout.at[pl.ds(pl.multiple_of(c * E + eb, 16), CH)], arow)

        def exg(i, _):
            a16 = arow[pl.ds(i * 16, 16)]
            e16 = jnp.exp(a16 - gmax)
            exrow[pl.ds(i * 16, 16)] = e16
            d16 = dstb[0, pl.ds(i * 16, 16)]
            plsc.addupdate_scatter(den, [d16], e16)
            return 0
        lax.fori_loop(0, CH // 16, exg, 0)
        cv.wait()

        def sgrp(g, _):
            ex16 = exrow[pl.ds(g * 16, 16)]

            def srow(r, _):
                e = g * 16 + r
                bc = bcast_lane(ex16, r)
                for jj in range(8):
                    qbuf[e, pl.ds(jj * 16, 16)] = (
                        qbuf[e, pl.ds(jj * 16, 16)] * bc)
                return 0
            lax.fori_loop(0, 16, srow, 0)
            return 0
        lax.fori_loop(0, CH // 16, sgrp, 0)
        pltpu.sync_copy(qbuf, agg_s.at[dstb.at[0]], add=True)
        return 0
    lax.fori_loop(0, NCH, p2_chunk, 0)
    pltpu.sync_copy(den, dens_s.at[pl.ds(pl.multiple_of(w * NP, 16), NP)])
    plsc.subcore_barrier()

    def dz(i, _):
        dacc[pl.ds(i * 16, 16)] = zero16
        return 0
    lax.fori_loop(0, NPT // 16, dz, 0)

    def dred(t, _):
        pltpu.sync_copy(dens_s.at[pl.ds(pl.multiple_of(t * NP + nbase, 16), NPT)], dtmp)

        def dadd(i, _):
            dacc[pl.ds(i * 16, 16)] = (dacc[pl.ds(i * 16, 16)]
                                       + dtmp[pl.ds(i * 16, 16)])
            return 0
        lax.fori_loop(0, NPT // 16, dadd, 0)
        return 0
    lax.fori_loop(0, 16, dred, 0)

    def outc(ch, _):
        rb = nbase + ch * 16
        pltpu.sync_copy(agg_s.at[pl.ds(pl.multiple_of(rb, 16), 16)], aggbuf)

        inv16 = 1.0 / (dacc[pl.ds(ch * 16, 16)] + 1e-16)

        def rown(r, _):
            bc = bcast_lane(inv16, r)
            for jj in range(8):
                aggbuf[r, pl.ds(jj * 16, 16)] = (
                    aggbuf[r, pl.ds(jj * 16, 16)] * bc)
            return 0
        lax.fori_loop(0, 16, rown, 0)
        pltpu.sync_copy(aggbuf, out.at[pl.ds(pl.multiple_of(c * NP + rb, 16), 16)])
        return 0
    lax.fori_loop(0, NPT // 16, outc, 0)


def _edge_call(qh, kh, vh, srcv, dstv):
    mesh = plsc.VectorSubcoreMesh(core_axis_name="c", subcore_axis_name="s")
    f = pl.kernel(
        _edge_body,
        out_type=(jax.ShapeDtypeStruct((H * NP, OC), jnp.float32),
                  jax.ShapeDtypeStruct((H * E,), jnp.float32)),
        mesh=mesh,
        compiler_params=pltpu.CompilerParams(needs_layout_passes=False),
        scratch_types=[
            pltpu.VMEM((CH,), jnp.int32),
            pltpu.VMEM((CH,), jnp.int32),
            pltpu.VMEM((1, CH), jnp.int32),
            pltpu.VMEM((CH,), jnp.int32),
            pltpu.VMEM((CH,), jnp.float32),
            pltpu.VMEM((CH, OC), jnp.float32),
            pltpu.VMEM((CH, OC), jnp.float32),
            pltpu.VMEM((CH,), jnp.float32),
            pltpu.VMEM((NP,), jnp.float32),
            pltpu.VMEM((NPT,), jnp.float32),
            pltpu.VMEM((NPT,), jnp.float32),
            pltpu.VMEM((16,), jnp.float32),
            pltpu.VMEM((256,), jnp.float32),
            pltpu.VMEM((16, OC), jnp.float32),
            pltpu.VMEM_SHARED((NP, OC), jnp.float32),
            pltpu.VMEM_SHARED((16 * NP,), jnp.float32),
            pltpu.VMEM_SHARED((256,), jnp.float32),
            pltpu.SemaphoreType.DMA,
            pltpu.SemaphoreType.DMA,
        ],
    )
    agg, _unused_alpha = f(qh, kh, vh, srcv, dstv)
    return agg.reshape(H, NP, OC)


def _tconv_sc(h, srcv, dstv, Wq, bq, Wk, bk, Wv, bv, Ws, bs):
    q = (h @ Wq + bq).reshape(2 * NP, OC)
    k = (h @ Wk + bk).reshape(2 * NP, OC)
    v = (h @ Wv + bv).reshape(2 * NP, OC)
    agg = _edge_call(q, k, v, srcv, dstv)
    aggc = jnp.concatenate([agg[0], agg[1]], axis=1)
    return aggc + h @ Ws + bs


def _ln(x, g, b):
    mu = jnp.mean(x, axis=-1, keepdims=True)
    var = jnp.var(x, axis=-1, keepdims=True)
    return (x - mu) / jnp.sqrt(var + EPS) * g + b


def _decoder_body(gf_ref, fcW_ref, fcb_ref, emb_ref, Wih_ref, Whh_ref,
                  bihh_ref, outW_ref, outb_ref, out_ref):
    gf = gf_ref[...]
    enc = jax.nn.relu(
        jnp.dot(gf, fcW_ref[...], preferred_element_type=jnp.float32)
        + fcb_ref[...])
    emb = emb_ref[...]
    Wih = Wih_ref[...]
    Whh = Whh_ref[...]
    bihh = bihh_ref[...]
    outW = outW_ref[...]
    outb = outb_ref[...]
    hs = enc
    cs = jnp.zeros_like(enc)
    inp = jnp.broadcast_to(emb[1], (B, ED))
    for t in range(T):
        gates = (jnp.dot(inp, Wih, preferred_element_type=jnp.float32)
                 + jnp.dot(hs, Whh, preferred_element_type=jnp.float32)
                 + bihh)
        i_ = gates[:, 0 * ED:1 * ED]
        f_ = gates[:, 1 * ED:2 * ED]
        g_ = gates[:, 2 * ED:3 * ED]
        o_ = gates[:, 3 * ED:4 * ED]
        cs = jax.nn.sigmoid(f_) * cs + jax.nn.sigmoid(i_) * jnp.tanh(g_)
        hs = jax.nn.sigmoid(o_) * jnp.tanh(cs)
        logits = (jnp.dot(hs, outW, preferred_element_type=jnp.float32)
                  + outb)
        out_ref[:, t, :] = logits
        m = jnp.max(logits, axis=-1, keepdims=True)
        iota = lax.broadcasted_iota(jnp.int32, (B, V), 1)
        tok = jnp.min(jnp.where(logits == m, iota, V), axis=-1,
                      keepdims=True)
        onehot = (lax.broadcasted_iota(jnp.int32, (B, V), 1)
                  == tok).astype(jnp.float32)
        inp = jnp.dot(onehot, emb, preferred_element_type=jnp.float32)


def _decode(gf, fcW, fcb, emb, Wih, Whh, bihh, outW, outb):
    return pl.pallas_call(
        _decoder_body,
        out_shape=jax.ShapeDtypeStruct((B, T, V), jnp.float32),
    )(gf, fcW, fcb, emb, Wih, Whh, bihh, outW, outb)


def kernel(x, edge_index, batch, gc_content, seq_length,
           Wq0, bq0, Wk0, bk0, Wv0, bv0, Ws0, bs0, g0, b0,
           Wq1, bq1, Wk1, bk1, Wv1, bv1, Ws1, bs1, g1, b1,
           Wq2, bq2, Wk2, bk2, Wv2, bv2, Ws2, bs2, g2, b2,
           fcW, fcb, emb, Wih, Whh, bih, bhh, outW, outb):
    srcv = edge_index[0]
    dstv = edge_index[1]
    xp = jnp.zeros((NP, 4), jnp.float32).at[:N].set(x)
    layers = [
        (Wq0, bq0, Wk0, bk0, Wv0, bv0, Ws0, bs0, g0, b0),
        (Wq1, bq1, Wk1, bk1, Wv1, bv1, Ws1, bs1, g1, b1),
        (Wq2, bq2, Wk2, bk2, Wv2, bv2, Ws2, bs2, g2, b2),
    ]
    h = xp
    for (Wq, bq, Wk, bk, Wv, bv, Ws, bs, g, b) in layers:
        h = jax.nn.relu(_tconv_sc(h, srcv, dstv,
                                  Wq, bq, Wk, bk, Wv, bv, Ws, bs))
        h = _ln(h, g, b)
    hn = h[:N]
    cnt = jax.ops.segment_sum(jnp.ones((N,), jnp.float32), batch,
                              num_segments=B)
    pooled = (jax.ops.segment_sum(hn, batch, num_segments=B)
              / jnp.maximum(cnt, 1.0)[:, None])
    gf = jnp.concatenate(
        [pooled, gc_content[:, None], seq_length[:, None]], axis=1)
    return _decode(gf, fcW, fcb, emb, Wih, Whh, bih + bhh, outW, outb)
